# baseline (speedup 1.0000x reference)
import functools

import jax
import jax.numpy as jnp
from jax import lax
from jax.experimental import pallas as pl
from jax.experimental.pallas import tpu as pltpu
from jax.experimental.pallas import tpu_sc as plsc

_NC = 2
_NS = 16
_NW = _NC * _NS
_LANES = 16


def _gather_body(e, c_gat, feat_hbm, obj_hbm, sub_hbm, nd_hbm,
                 sub_rows_hbm, obj_rows_hbm, norm_hbm,
                 ndo_v, nds_v, oi_v, si_v, orows_v, srows_v, norm_v,
                 gsem0, gsem1, ndsem0, ndsem1, wsem0, wsem1):
    wid = lax.axis_index("s") * _NC + lax.axis_index("c")
    ew = e // _NW
    nchunk = ew // c_gat
    nbuf = 2
    gsem = (gsem0, gsem1)
    ndsem = (ndsem0, ndsem1)
    wsem = (wsem0, wsem1)

    def load_idx_and_fire(i, b):
        base = wid * ew + i * c_gat
        pltpu.sync_copy(obj_hbm.at[pl.ds(base, c_gat)], oi_v.at[b])
        pltpu.sync_copy(sub_hbm.at[pl.ds(base, c_gat)], si_v.at[b])
        pltpu.async_copy(feat_hbm.at[oi_v.at[b]], orows_v.at[b], gsem[b])
        pltpu.async_copy(feat_hbm.at[si_v.at[b]], srows_v.at[b], gsem[b])
        pltpu.async_copy(nd_hbm.at[oi_v.at[b]], ndo_v.at[b], ndsem[b])
        pltpu.async_copy(nd_hbm.at[si_v.at[b]], nds_v.at[b], ndsem[b])

    def drain_and_write(i, b):
        base = wid * ew + i * c_gat
        pltpu.make_async_copy(nd_hbm.at[oi_v.at[b]], ndo_v.at[b], ndsem[b]).wait()
        pltpu.make_async_copy(nd_hbm.at[si_v.at[b]], nds_v.at[b], ndsem[b]).wait()
        for j in range(c_gat // _LANES):
            sl = pl.ds(j * _LANES, _LANES)
            norm_v[b, sl] = ndo_v[b, sl] * nds_v[b, sl]
        pltpu.make_async_copy(feat_hbm.at[oi_v.at[b]], orows_v.at[b], gsem[b]).wait()
        pltpu.make_async_copy(feat_hbm.at[si_v.at[b]], srows_v.at[b], gsem[b]).wait()
        pltpu.async_copy(orows_v.at[b], obj_rows_hbm.at[pl.ds(base, c_gat)], wsem[b])
        pltpu.async_copy(srows_v.at[b], sub_rows_hbm.at[pl.ds(base, c_gat)], wsem[b])
        pltpu.async_copy(norm_v.at[b], norm_hbm.at[pl.ds(base, c_gat)], wsem[b])

    def wait_writes(i, b):
        base = wid * ew + i * c_gat
        pltpu.make_async_copy(orows_v.at[b], obj_rows_hbm.at[pl.ds(base, c_gat)], wsem[b]).wait()
        pltpu.make_async_copy(srows_v.at[b], sub_rows_hbm.at[pl.ds(base, c_gat)], wsem[b]).wait()
        pltpu.make_async_copy(norm_v.at[b], norm_hbm.at[pl.ds(base, c_gat)], wsem[b]).wait()

    load_idx_and_fire(0, 0)
    load_idx_and_fire(1, 1)

    def step(io, _):
        for b in range(nbuf):
            i = io * nbuf + b
            drain_and_write(i, b)
            wait_writes(i, b)

            @pl.when(i + nbuf < nchunk)
            def _():
                load_idx_and_fire(i + nbuf, b)
        return _

    lax.fori_loop(0, nchunk // nbuf, step, None)


def _gate_body(n, sub_ref, obj_ref, nrm_ref, oid_ref, sid_ref,
               gam_ref, bet_ref, w_ref, fcb_ref, g_ref, flat_ref):
    h = sub_ref.shape[1]
    inv = 1.0 / (2 * h)
    x1 = sub_ref[...].astype(jnp.float32)
    x2 = obj_ref[...].astype(jnp.float32)
    s = jnp.sum(x1, axis=1) + jnp.sum(x2, axis=1)
    q = jnp.sum(x1 * x1, axis=1) + jnp.sum(x2 * x2, axis=1)
    mu = s * inv
    var = q * inv - mu * mu
    r = lax.rsqrt(var + 1e-5)
    xm1 = (x1 - mu[:, None]) * r[:, None]
    xm2 = (x2 - mu[:, None]) * r[:, None]
    h1 = jnp.maximum(xm1 * gam_ref[0, :h][None, :] + bet_ref[0, :h][None, :], 0.0)
    h2 = jnp.maximum(xm2 * gam_ref[0, h:][None, :] + bet_ref[0, h:][None, :], 0.0)
    t = (jnp.dot(h1, w_ref[:h, :], preferred_element_type=jnp.float32)
         + jnp.dot(h2, w_ref[h:, :], preferred_element_type=jnp.float32))
    t = t + fcb_ref[0, 0]
    nm = nrm_ref[...]
    nm = jnp.where(nm > 10000.0, 0.0, nm)
    g_ref[...] = jnp.tanh(t) * nm
    flat_ref[...] = sid_ref[...] * n + oid_ref[...]


def _scatter_body(e, c_sc, flat_hbm, g_hbm, a_hbm, fl_v, g_v, sem):
    wid = lax.axis_index("s") * _NC + lax.axis_index("c")
    nchunk = (e // c_sc) // _NW
    row0 = wid * nchunk
    pltpu.sync_copy(flat_hbm.at[pl.ds(row0, nchunk)], fl_v)
    pltpu.sync_copy(g_hbm.at[pl.ds(row0, nchunk)], g_v)

    def fire(io, _):
        for jj in range(8):
            j = io * 8 + jj
            pltpu.async_copy(g_v.at[j], a_hbm.at[fl_v.at[j]], sem)
        return _

    lax.fori_loop(0, nchunk // 8, fire, None)

    def drain(io, _):
        for _jj in range(8):
            pltpu.make_async_copy(g_v.at[0], a_hbm.at[fl_v.at[0]], sem).wait()
        return _

    lax.fori_loop(0, nchunk // 8, drain, None)


def _matmul_body(a_ref, f_ref, o_ref):
    o_ref[...] = jnp.dot(a_ref[...].astype(jnp.bfloat16),
                         f_ref[...].astype(jnp.bfloat16),
                         preferred_element_type=jnp.float32)


def kernel(inst_feature, norm_degree, aggregator_matrix, rel_pair_index,
           ln_gamma, ln_beta, fc_w, fc_b):
    n, h = inst_feature.shape
    e = rel_pair_index.shape[0]
    nslice = 4
    es = e // nslice
    c_gat = 64
    c_sc = 128
    hw = h // 2
    assert es % (_NW * c_gat) == 0 and h % 128 == 0

    obj_idx = rel_pair_index[:, 0].astype(jnp.int32)
    sub_idx = rel_pair_index[:, 1].astype(jnp.int32)
    feat_b = lax.bitcast_convert_type(
        inst_feature.astype(jnp.bfloat16).reshape(n, hw, 2), jnp.int32)

    mesh = plsc.VectorSubcoreMesh(core_axis_name="c", subcore_axis_name="s")

    gather_k = functools.partial(
        pl.kernel,
        out_type=(
            jax.ShapeDtypeStruct((es, hw), jnp.int32),
            jax.ShapeDtypeStruct((es, hw), jnp.int32),
            jax.ShapeDtypeStruct((es,), jnp.float32),
        ),
        mesh=mesh,
        scratch_types=[
            pltpu.VMEM((2, c_gat), jnp.float32),
            pltpu.VMEM((2, c_gat), jnp.float32),
            pltpu.VMEM((2, c_gat), jnp.int32),
            pltpu.VMEM((2, c_gat), jnp.int32),
            pltpu.VMEM((2, c_gat, hw), jnp.int32),
            pltpu.VMEM((2, c_gat, hw), jnp.int32),
            pltpu.VMEM((2, c_gat), jnp.float32),
            pltpu.SemaphoreType.DMA,
            pltpu.SemaphoreType.DMA,
            pltpu.SemaphoreType.DMA,
            pltpu.SemaphoreType.DMA,
            pltpu.SemaphoreType.DMA,
            pltpu.SemaphoreType.DMA,
        ],
    )(functools.partial(_gather_body, es, c_gat))

    b = 1024
    gate_k = pl.pallas_call(
        functools.partial(_gate_body, n),
        grid=(es // b,),
        in_specs=[
            pl.BlockSpec((b, h), lambda i: (i, 0)),
            pl.BlockSpec((b, h), lambda i: (i, 0)),
            pl.BlockSpec((b, 1), lambda i: (i, 0)),
            pl.BlockSpec((b, 1), lambda i: (i, 0)),
            pl.BlockSpec((b, 1), lambda i: (i, 0)),
            pl.BlockSpec((1, 2 * h), lambda i: (0, 0)),
            pl.BlockSpec((1, 2 * h), lambda i: (0, 0)),
            pl.BlockSpec((2 * h, 1), lambda i: (0, 0)),
            pl.BlockSpec((1, 1), lambda i: (0, 0)),
        ],
        out_specs=[
            pl.BlockSpec((b, 1), lambda i: (i, 0)),
            pl.BlockSpec((b, 1), lambda i: (i, 0)),
        ],
        out_shape=[
            jax.ShapeDtypeStruct((es, 1), jnp.float32),
            jax.ShapeDtypeStruct((es, 1), jnp.int32),
        ],
    )

    a_ref = jax.new_ref(jnp.zeros((n * n,), jnp.float32))
    nchunk_w = (es // c_sc) // _NW
    scatter_k = functools.partial(
        pl.kernel,
        out_type=(),
        mesh=mesh,
        scratch_types=[
            pltpu.VMEM((nchunk_w, c_sc), jnp.int32),
            pltpu.VMEM((nchunk_w, c_sc), jnp.float32),
            pltpu.SemaphoreType.DMA,
        ],
    )(functools.partial(_scatter_body, es, c_sc))

    g_slices = []
    for k in range(nslice):
        oi_k = lax.slice_in_dim(obj_idx, k * es, (k + 1) * es)
        si_k = lax.slice_in_dim(sub_idx, k * es, (k + 1) * es)
        sub_rows, obj_rows, norm = gather_k(feat_b, oi_k, si_k, norm_degree)
        sub_bf = lax.bitcast_convert_type(sub_rows, jnp.bfloat16).reshape(es, h)
        obj_bf = lax.bitcast_convert_type(obj_rows, jnp.bfloat16).reshape(es, h)
        g2d, flat2d = gate_k(
            sub_bf, obj_bf,
            norm.reshape(es, 1), oi_k.reshape(es, 1), si_k.reshape(es, 1),
            ln_gamma.reshape(1, 2 * h), ln_beta.reshape(1, 2 * h),
            fc_w.reshape(2 * h, 1), fc_b.reshape(1, 1))
        scatter_k(flat2d.reshape(es // c_sc, c_sc),
                  g2d.reshape(es // c_sc, c_sc), a_ref)
        g_slices.append(g2d.reshape(es))
    g = jnp.concatenate(g_slices)
    a_mat = a_ref[...].reshape(n, n)

    bm = 512
    mm_k = pl.pallas_call(
        _matmul_body,
        grid=(n // bm,),
        in_specs=[
            pl.BlockSpec((bm, n), lambda i: (i, 0)),
            pl.BlockSpec((n, h), lambda i: (0, 0)),
        ],
        out_specs=pl.BlockSpec((bm, h), lambda i: (i, 0)),
        out_shape=jax.ShapeDtypeStruct((n, h), jnp.float32),
    )
    aggregator_feature = mm_k(a_mat, inst_feature)
    return (aggregator_feature, g)


# speedup vs baseline: 3.0705x; 3.0705x over previous
import functools

import jax
import jax.numpy as jnp
from jax import lax
from jax.experimental import pallas as pl
from jax.experimental.pallas import tpu as pltpu
from jax.experimental.pallas import tpu_sc as plsc

_NC = 2
_NS = 16
_NW = _NC * _NS
_LANES = 16


def _gather_body(e, c_gat, feat_hbm, obj_hbm, sub_hbm, nd_hbm,
                 sub_rows_hbm, obj_rows_hbm, norm_hbm,
                 ndo_v, nds_v, oi_v, si_v, orows_v, srows_v, norm_v,
                 gsem0, gsem1, ndsem0, ndsem1, wsem0, wsem1):
    wid = lax.axis_index("s") * _NC + lax.axis_index("c")
    ew = e // _NW
    nchunk = ew // c_gat
    nbuf = 2
    gsem = (gsem0, gsem1)
    ndsem = (ndsem0, ndsem1)
    wsem = (wsem0, wsem1)

    def load_idx_and_fire(i, b):
        base = wid * ew + i * c_gat
        pltpu.sync_copy(obj_hbm.at[pl.ds(base, c_gat)], oi_v.at[b])
        pltpu.sync_copy(sub_hbm.at[pl.ds(base, c_gat)], si_v.at[b])
        pltpu.async_copy(feat_hbm.at[oi_v.at[b]], orows_v.at[b], gsem[b])
        pltpu.async_copy(feat_hbm.at[si_v.at[b]], srows_v.at[b], gsem[b])
        pltpu.async_copy(nd_hbm.at[oi_v.at[b]], ndo_v.at[b], ndsem[b])
        pltpu.async_copy(nd_hbm.at[si_v.at[b]], nds_v.at[b], ndsem[b])

    def drain_and_write(i, b):
        base = wid * ew + i * c_gat
        pltpu.make_async_copy(nd_hbm.at[oi_v.at[b]], ndo_v.at[b], ndsem[b]).wait()
        pltpu.make_async_copy(nd_hbm.at[si_v.at[b]], nds_v.at[b], ndsem[b]).wait()
        for j in range(c_gat // _LANES):
            sl = pl.ds(j * _LANES, _LANES)
            norm_v[b, sl] = ndo_v[b, sl] * nds_v[b, sl]
        pltpu.make_async_copy(feat_hbm.at[oi_v.at[b]], orows_v.at[b], gsem[b]).wait()
        pltpu.make_async_copy(feat_hbm.at[si_v.at[b]], srows_v.at[b], gsem[b]).wait()
        pltpu.async_copy(orows_v.at[b], obj_rows_hbm.at[pl.ds(base, c_gat)], wsem[b])
        pltpu.async_copy(srows_v.at[b], sub_rows_hbm.at[pl.ds(base, c_gat)], wsem[b])
        pltpu.async_copy(norm_v.at[b], norm_hbm.at[pl.ds(base, c_gat)], wsem[b])

    def wait_writes(i, b):
        base = wid * ew + i * c_gat
        pltpu.make_async_copy(orows_v.at[b], obj_rows_hbm.at[pl.ds(base, c_gat)], wsem[b]).wait()
        pltpu.make_async_copy(srows_v.at[b], sub_rows_hbm.at[pl.ds(base, c_gat)], wsem[b]).wait()
        pltpu.make_async_copy(norm_v.at[b], norm_hbm.at[pl.ds(base, c_gat)], wsem[b]).wait()

    load_idx_and_fire(0, 0)
    load_idx_and_fire(1, 1)

    def step(io, _):
        for b in range(nbuf):
            i = io * nbuf + b
            drain_and_write(i, b)
            wait_writes(i, b)

            @pl.when(i + nbuf < nchunk)
            def _():
                load_idx_and_fire(i + nbuf, b)
        return _

    lax.fori_loop(0, nchunk // nbuf, step, None)


def _gate_body(n, sub_ref, obj_ref, nrm_ref, oid_ref, sid_ref,
               p_ref, fcb_ref, g_ref, flat_ref):
    hw = sub_ref.shape[1]
    inv = 1.0 / (4 * hw)
    x1 = sub_ref[...]
    x2 = obj_ref[...]
    hi_mask = jnp.int32(-65536)
    se = lax.bitcast_convert_type(lax.shift_left(x1, 16), jnp.float32)
    so = lax.bitcast_convert_type(x1 & hi_mask, jnp.float32)
    oe = lax.bitcast_convert_type(lax.shift_left(x2, 16), jnp.float32)
    oo = lax.bitcast_convert_type(x2 & hi_mask, jnp.float32)
    s = (jnp.sum(se, axis=1) + jnp.sum(so, axis=1)
         + jnp.sum(oe, axis=1) + jnp.sum(oo, axis=1))
    q = (jnp.sum(se * se, axis=1) + jnp.sum(so * so, axis=1)
         + jnp.sum(oe * oe, axis=1) + jnp.sum(oo * oo, axis=1))
    mu = s * inv
    var = q * inv - mu * mu
    r = lax.rsqrt(var + 1e-5)
    mu2 = mu[:, None]
    r2 = r[:, None]
    t = fcb_ref[0, 0]
    for i, x in enumerate((se, so, oe, oo)):
        gam = p_ref[i, :][None, :]
        bet = p_ref[4 + i, :][None, :]
        w = p_ref[8 + i, :][None, :]
        hx = jnp.maximum((x - mu2) * r2 * gam + bet, 0.0)
        t = t + lax.dot_general(hx, w, (((1,), (1,)), ((), ())),
                                preferred_element_type=jnp.float32)
    nm = nrm_ref[...]
    nm = jnp.where(nm > 10000.0, 0.0, nm)
    g_ref[...] = jnp.tanh(t) * nm
    flat_ref[...] = sid_ref[...] * n + oid_ref[...]


def _scatter_body(e, c_sc, flat_hbm, g_hbm, a_hbm, fl_v, g_v, sem):
    wid = lax.axis_index("s") * _NC + lax.axis_index("c")
    nchunk = (e // c_sc) // _NW
    row0 = wid * nchunk
    pltpu.sync_copy(flat_hbm.at[pl.ds(row0, nchunk)], fl_v)
    pltpu.sync_copy(g_hbm.at[pl.ds(row0, nchunk)], g_v)

    def fire(io, _):
        for jj in range(8):
            j = io * 8 + jj
            pltpu.async_copy(g_v.at[j], a_hbm.at[fl_v.at[j]], sem)
        return _

    lax.fori_loop(0, nchunk // 8, fire, None)

    def drain(io, _):
        for _jj in range(8):
            pltpu.make_async_copy(g_v.at[0], a_hbm.at[fl_v.at[0]], sem).wait()
        return _

    lax.fori_loop(0, nchunk // 8, drain, None)


def _matmul_body(a_ref, f_ref, o_ref):
    o_ref[...] = jnp.dot(a_ref[...].astype(jnp.bfloat16),
                         f_ref[...].astype(jnp.bfloat16),
                         preferred_element_type=jnp.float32)


def kernel(inst_feature, norm_degree, aggregator_matrix, rel_pair_index,
           ln_gamma, ln_beta, fc_w, fc_b):
    n, h = inst_feature.shape
    e = rel_pair_index.shape[0]
    nslice = 4
    es = e // nslice
    c_gat = 64
    c_sc = 128
    hw = h // 2
    assert es % (_NW * c_gat) == 0 and h % 128 == 0

    obj_idx = rel_pair_index[:, 0].astype(jnp.int32)
    sub_idx = rel_pair_index[:, 1].astype(jnp.int32)
    feat_b = lax.bitcast_convert_type(
        inst_feature.astype(jnp.bfloat16).reshape(n, hw, 2), jnp.int32)

    mesh = plsc.VectorSubcoreMesh(core_axis_name="c", subcore_axis_name="s")

    gather_k = functools.partial(
        pl.kernel,
        out_type=(
            jax.ShapeDtypeStruct((es, hw), jnp.int32),
            jax.ShapeDtypeStruct((es, hw), jnp.int32),
            jax.ShapeDtypeStruct((es,), jnp.float32),
        ),
        mesh=mesh,
        scratch_types=[
            pltpu.VMEM((2, c_gat), jnp.float32),
            pltpu.VMEM((2, c_gat), jnp.float32),
            pltpu.VMEM((2, c_gat), jnp.int32),
            pltpu.VMEM((2, c_gat), jnp.int32),
            pltpu.VMEM((2, c_gat, hw), jnp.int32),
            pltpu.VMEM((2, c_gat, hw), jnp.int32),
            pltpu.VMEM((2, c_gat), jnp.float32),
            pltpu.SemaphoreType.DMA,
            pltpu.SemaphoreType.DMA,
            pltpu.SemaphoreType.DMA,
            pltpu.SemaphoreType.DMA,
            pltpu.SemaphoreType.DMA,
            pltpu.SemaphoreType.DMA,
        ],
    )(functools.partial(_gather_body, es, c_gat))

    rows = []
    for arr in (ln_gamma, ln_beta, fc_w.reshape(2 * h)):
        for half in (0, h):
            rows.append(arr[half:half + h:2])
            rows.append(arr[half + 1:half + h:2])
    params = jnp.concatenate(
        [jnp.stack(rows), jnp.zeros((4, hw), jnp.float32)])

    b = 1024
    gate_k = pl.pallas_call(
        functools.partial(_gate_body, n),
        grid=(es // b,),
        in_specs=[
            pl.BlockSpec((b, hw), lambda i: (i, 0)),
            pl.BlockSpec((b, hw), lambda i: (i, 0)),
            pl.BlockSpec((b, 1), lambda i: (i, 0)),
            pl.BlockSpec((b, 1), lambda i: (i, 0)),
            pl.BlockSpec((b, 1), lambda i: (i, 0)),
            pl.BlockSpec((16, hw), lambda i: (0, 0)),
            pl.BlockSpec((1, 1), lambda i: (0, 0)),
        ],
        out_specs=[
            pl.BlockSpec((b, 1), lambda i: (i, 0)),
            pl.BlockSpec((b, 1), lambda i: (i, 0)),
        ],
        out_shape=[
            jax.ShapeDtypeStruct((es, 1), jnp.float32),
            jax.ShapeDtypeStruct((es, 1), jnp.int32),
        ],
    )

    a_ref = jax.new_ref(jnp.zeros((n * n,), jnp.float32))
    nchunk_w = (es // c_sc) // _NW
    scatter_k = functools.partial(
        pl.kernel,
        out_type=(),
        mesh=mesh,
        scratch_types=[
            pltpu.VMEM((nchunk_w, c_sc), jnp.int32),
            pltpu.VMEM((nchunk_w, c_sc), jnp.float32),
            pltpu.SemaphoreType.DMA,
        ],
    )(functools.partial(_scatter_body, es, c_sc))

    g_slices = []
    for k in range(nslice):
        oi_k = lax.slice_in_dim(obj_idx, k * es, (k + 1) * es)
        si_k = lax.slice_in_dim(sub_idx, k * es, (k + 1) * es)
        sub_rows, obj_rows, norm = gather_k(feat_b, oi_k, si_k, norm_degree)
        g2d, flat2d = gate_k(
            sub_rows, obj_rows,
            norm.reshape(es, 1), oi_k.reshape(es, 1), si_k.reshape(es, 1),
            params, fc_b.reshape(1, 1))
        scatter_k(flat2d.reshape(es // c_sc, c_sc),
                  g2d.reshape(es // c_sc, c_sc), a_ref)
        g_slices.append(g2d.reshape(es))
    g = jnp.concatenate(g_slices)
    a_mat = a_ref[...].reshape(n, n)

    bm = 512
    mm_k = pl.pallas_call(
        _matmul_body,
        grid=(n // bm,),
        in_specs=[
            pl.BlockSpec((bm, n), lambda i: (i, 0)),
            pl.BlockSpec((n, h), lambda i: (0, 0)),
        ],
        out_specs=pl.BlockSpec((bm, h), lambda i: (i, 0)),
        out_shape=jax.ShapeDtypeStruct((n, h), jnp.float32),
    )
    aggregator_feature = mm_k(a_mat, inst_feature)
    return (aggregator_feature, g)
